# baseline (speedup 1.0000x reference)
import functools

import jax
import jax.numpy as jnp
from jax import lax
from jax.experimental import pallas as pl
from jax.experimental.pallas import tpu as pltpu
from jax.experimental.pallas import tpu_sc as plsc

N, E, D = 10000, 320000, 128
NC, NS = 2, 16
NW = NC * NS
CH = 128
M = 2 * E
NCHUNK = (-(-M // (NW * CH)) + 7) // 8 * 8
M_PAD = NW * CH * NCHUNK
NCH_TOT = M_PAD // CH
R_ACC = 10240
ZROWS = R_ACC // NS

BLK = 1000


def _dense1_body(x_ref, win_ref, wout_ref, wloop_ref, bin_ref, bout_ref,
                 wg_ref, bg_ref, gated_ref, loop_ref):
    xb = x_ref[...]
    wg = wg_ref[...]
    bg = bg_ref[...]

    def gate(k):
        s = jnp.sum(xb * wg[k:k + 1, :], axis=1, keepdims=True) + bg[:, k:k + 1]
        return 1.0 / (1.0 + jnp.exp(-s))

    it = jnp.dot(xb, win_ref[...], preferred_element_type=jnp.float32) + bin_ref[...]
    gated_ref[0] = it * gate(0)
    ot = jnp.dot(xb, wout_ref[...], preferred_element_type=jnp.float32) + bout_ref[...]
    gated_ref[1] = ot * gate(1)
    lt = jnp.dot(xb, wloop_ref[...], preferred_element_type=jnp.float32)
    loop_ref[...] = lt * gate(2)


def _combine_body(p_ref, loop_ref, out_ref):
    out_ref[...] = jnp.maximum(p_ref[0] + p_ref[1] + loop_ref[...], 0.0)


def _sc_scatter_body(table_hbm, qidx_hbm, out_hbm,
                     q0, q1, rows0, rows1, acc, gsem0, gsem1):
    cid = lax.axis_index("c")
    sid = lax.axis_index("s")
    wid = cid * NS + sid
    qv = (q0, q1)
    rows = (rows0, rows1)
    gsem = (gsem0, gsem1)

    zero16 = jnp.zeros((16,), jnp.float32)

    def zloop(i, _):
        r = i // 8
        c = (i % 8) * 16
        rows0[r, pl.ds(c, 16)] = zero16
        return 0

    lax.fori_loop(0, CH * 8, zloop, 0)
    for k in range(ZROWS // CH):
        pltpu.sync_copy(rows0, acc.at[pl.ds(sid * ZROWS + k * CH, CH)])
    plsc.subcore_barrier()

    base = wid * NCHUNK

    def gather(m, p):
        pltpu.async_copy(table_hbm.at[qv[m].at[0]], rows[p], gsem[p])

    def gwait(p):
        pltpu.make_async_copy(table_hbm.at[qv[0].at[0]], rows[p], gsem[p]).wait()

    def step(c, p):
        q = 1 - p
        gwait(p)
        pltpu.sync_copy(qidx_hbm.at[base + c + 1], qv[q])
        gather(q, q)
        pltpu.sync_copy(rows[p], acc.at[qv[p].at[1]], add=True)

    pltpu.sync_copy(qidx_hbm.at[base], qv[0])
    gather(0, 0)

    def group(k, _):
        step(2 * k, 0)
        step(2 * k + 1, 1)
        return 0

    lax.fori_loop(0, NCHUNK // 2 - 1, group, 0)

    gwait(0)
    pltpu.sync_copy(qidx_hbm.at[base + NCHUNK - 1], qv[1])
    gather(1, 1)
    pltpu.sync_copy(rows[0], acc.at[qv[0].at[1]], add=True)
    gwait(1)
    pltpu.sync_copy(rows[1], acc.at[qv[1].at[1]], add=True)
    plsc.subcore_barrier()

    pltpu.sync_copy(acc.at[pl.ds(sid * ZROWS, ZROWS)],
                    out_hbm.at[cid, pl.ds(sid * ZROWS, ZROWS)])


@jax.jit
def _run(x, edge_index, w_in, b_in, w_out, b_out, w_loop, w_gin, b_gin,
         w_gout, b_gout, w_gloop):
    src = edge_index[0].astype(jnp.int32)
    dst = edge_index[1].astype(jnp.int32)
    pad = M_PAD - M
    gidx = jnp.concatenate([src, dst + N, jnp.zeros((pad,), jnp.int32)])
    sidx = jnp.concatenate(
        [dst, src, N + (jnp.arange(pad, dtype=jnp.int32) % (R_ACC - N))])
    qidx = jnp.stack([gidx.reshape(NCH_TOT, CH), sidx.reshape(NCH_TOT, CH)],
                     axis=1)

    wg = jnp.concatenate([w_gin.T, w_gout.T, w_gloop.T], axis=0)
    bg = jnp.stack([b_gin[0], b_gout[0], jnp.float32(0.0)]).reshape(1, 3)

    gated, loop_act = pl.pallas_call(
        _dense1_body,
        grid=(N // BLK,),
        in_specs=[
            pl.BlockSpec((BLK, D), lambda i: (i, 0)),
            pl.BlockSpec((D, D), lambda i: (0, 0)),
            pl.BlockSpec((D, D), lambda i: (0, 0)),
            pl.BlockSpec((D, D), lambda i: (0, 0)),
            pl.BlockSpec((1, D), lambda i: (0, 0)),
            pl.BlockSpec((1, D), lambda i: (0, 0)),
            pl.BlockSpec((3, D), lambda i: (0, 0)),
            pl.BlockSpec((1, 3), lambda i: (0, 0)),
        ],
        out_specs=[
            pl.BlockSpec((2, BLK, D), lambda i: (0, i, 0)),
            pl.BlockSpec((BLK, D), lambda i: (i, 0)),
        ],
        out_shape=[
            jax.ShapeDtypeStruct((2, N, D), jnp.float32),
            jax.ShapeDtypeStruct((N, D), jnp.float32),
        ],
    )(x, w_in, w_out, w_loop, b_in.reshape(1, D), b_out.reshape(1, D), wg, bg)

    table = gated.reshape(2 * N, D)

    mesh = plsc.VectorSubcoreMesh(core_axis_name="c", subcore_axis_name="s")
    partials = pl.kernel(
        _sc_scatter_body,
        out_type=jax.ShapeDtypeStruct((NC, R_ACC, D), jnp.float32),
        mesh=mesh,
        scratch_types=(
            [pltpu.VMEM((2, CH), jnp.int32)] * 2
            + [pltpu.VMEM((CH, D), jnp.float32)] * 2
            + [pltpu.VMEM_SHARED((R_ACC, D), jnp.float32)]
            + [pltpu.SemaphoreType.DMA] * 2
        ),
    )(table, qidx)

    out = pl.pallas_call(
        _combine_body,
        grid=(N // BLK,),
        in_specs=[
            pl.BlockSpec((2, BLK, D), lambda i: (0, i, 0)),
            pl.BlockSpec((BLK, D), lambda i: (i, 0)),
        ],
        out_specs=pl.BlockSpec((BLK, D), lambda i: (i, 0)),
        out_shape=jax.ShapeDtypeStruct((N, D), jnp.float32),
    )(partials, loop_act)
    return out


def kernel(x, edge_index, w_in, b_in, w_out, b_out, w_loop, w_gin, b_gin,
           w_gout, b_gout, w_gloop):
    return _run(x, edge_index, w_in, b_in, w_out, b_out, w_loop, w_gin, b_gin,
                w_gout, b_gout, w_gloop)


# speedup vs baseline: 1.8108x; 1.8108x over previous
import functools

import jax
import jax.numpy as jnp
from jax import lax
from jax.experimental import pallas as pl
from jax.experimental.pallas import tpu as pltpu
from jax.experimental.pallas import tpu_sc as plsc

N, E, D = 10000, 320000, 128
NC, NS = 2, 16
NW = NC * NS
CH = 128
M = 2 * E
NCHUNK = -(-M // (NW * CH))
M_PAD = NW * CH * NCHUNK
NCH_TOT = M_PAD // CH
R_ACC = 10240
ZROWS = R_ACC // NS

BLK = 1000


def _dense1_body(x_ref, win_ref, wout_ref, wloop_ref, bin_ref, bout_ref,
                 wg_ref, bg_ref, gated_ref, loop_ref):
    xb = x_ref[...]
    wg = wg_ref[...]
    bg = bg_ref[...]

    def gate(k):
        s = jnp.sum(xb * wg[k:k + 1, :], axis=1, keepdims=True) + bg[:, k:k + 1]
        return 1.0 / (1.0 + jnp.exp(-s))

    it = jnp.dot(xb, win_ref[...], preferred_element_type=jnp.float32) + bin_ref[...]
    gated_ref[0] = it * gate(0)
    ot = jnp.dot(xb, wout_ref[...], preferred_element_type=jnp.float32) + bout_ref[...]
    gated_ref[1] = ot * gate(1)
    lt = jnp.dot(xb, wloop_ref[...], preferred_element_type=jnp.float32)
    loop_ref[...] = lt * gate(2)


def _combine_body(p_ref, loop_ref, out_ref):
    out_ref[...] = jnp.maximum(p_ref[0] + p_ref[1] + loop_ref[...], 0.0)


def _sc_scatter_body(table_hbm, gidx_hbm, sidx_hbm, out_hbm,
                     gi_v, si_v, rows0, acc, gsem0):
    cid = lax.axis_index("c")
    sid = lax.axis_index("s")
    wid = cid * NS + sid

    zero16 = jnp.zeros((16,), jnp.float32)

    def zloop(i, _):
        r = i // 8
        c = (i % 8) * 16
        rows0[r, pl.ds(c, 16)] = zero16
        return 0

    lax.fori_loop(0, CH * 8, zloop, 0)
    for k in range(ZROWS // CH):
        pltpu.sync_copy(rows0, acc.at[pl.ds(sid * ZROWS + k * CH, CH)])
    plsc.subcore_barrier()

    def chunk(c, _):
        ch = wid * NCHUNK + c
        pltpu.sync_copy(gidx_hbm.at[ch], gi_v)
        pltpu.sync_copy(sidx_hbm.at[ch], si_v)
        pltpu.async_copy(table_hbm.at[gi_v], rows0, gsem0).wait()
        pltpu.sync_copy(rows0, acc.at[si_v], add=True)
        return 0

    lax.fori_loop(0, NCHUNK, chunk, 0)
    plsc.subcore_barrier()

    pltpu.sync_copy(acc.at[pl.ds(sid * ZROWS, ZROWS)],
                    out_hbm.at[cid, pl.ds(sid * ZROWS, ZROWS)])


@jax.jit
def _run(x, edge_index, w_in, b_in, w_out, b_out, w_loop, w_gin, b_gin,
         w_gout, b_gout, w_gloop):
    src = edge_index[0].astype(jnp.int32)
    dst = edge_index[1].astype(jnp.int32)
    pad = M_PAD - M
    gidx = jnp.concatenate([src, dst + N, jnp.zeros((pad,), jnp.int32)])
    sidx = jnp.concatenate(
        [dst, src, N + (jnp.arange(pad, dtype=jnp.int32) % (R_ACC - N))])
    gidx = gidx.reshape(NCH_TOT, CH)
    sidx = sidx.reshape(NCH_TOT, CH)

    wg = jnp.concatenate([w_gin.T, w_gout.T, w_gloop.T], axis=0)
    bg = jnp.stack([b_gin[0], b_gout[0], jnp.float32(0.0)]).reshape(1, 3)

    gated, loop_act = pl.pallas_call(
        _dense1_body,
        grid=(N // BLK,),
        in_specs=[
            pl.BlockSpec((BLK, D), lambda i: (i, 0)),
            pl.BlockSpec((D, D), lambda i: (0, 0)),
            pl.BlockSpec((D, D), lambda i: (0, 0)),
            pl.BlockSpec((D, D), lambda i: (0, 0)),
            pl.BlockSpec((1, D), lambda i: (0, 0)),
            pl.BlockSpec((1, D), lambda i: (0, 0)),
            pl.BlockSpec((3, D), lambda i: (0, 0)),
            pl.BlockSpec((1, 3), lambda i: (0, 0)),
        ],
        out_specs=[
            pl.BlockSpec((2, BLK, D), lambda i: (0, i, 0)),
            pl.BlockSpec((BLK, D), lambda i: (i, 0)),
        ],
        out_shape=[
            jax.ShapeDtypeStruct((2, N, D), jnp.float32),
            jax.ShapeDtypeStruct((N, D), jnp.float32),
        ],
    )(x, w_in, w_out, w_loop, b_in.reshape(1, D), b_out.reshape(1, D), wg, bg)

    table = gated.reshape(2 * N, D)

    mesh = plsc.VectorSubcoreMesh(core_axis_name="c", subcore_axis_name="s")
    partials = pl.kernel(
        _sc_scatter_body,
        out_type=jax.ShapeDtypeStruct((NC, R_ACC, D), jnp.float32),
        mesh=mesh,
        scratch_types=(
            [pltpu.VMEM((CH,), jnp.int32)] * 2
            + [pltpu.VMEM((CH, D), jnp.float32)]
            + [pltpu.VMEM_SHARED((R_ACC, D), jnp.float32)]
            + [pltpu.SemaphoreType.DMA]
        ),
    )(table, gidx, sidx)

    out = pl.pallas_call(
        _combine_body,
        grid=(N // BLK,),
        in_specs=[
            pl.BlockSpec((2, BLK, D), lambda i: (0, i, 0)),
            pl.BlockSpec((BLK, D), lambda i: (i, 0)),
        ],
        out_specs=pl.BlockSpec((BLK, D), lambda i: (i, 0)),
        out_shape=jax.ShapeDtypeStruct((N, D), jnp.float32),
    )(partials, loop_act)
    return out


def kernel(x, edge_index, w_in, b_in, w_out, b_out, w_loop, w_gin, b_gin,
           w_gout, b_gout, w_gloop):
    return _run(x, edge_index, w_in, b_in, w_out, b_out, w_loop, w_gin, b_gin,
                w_gout, b_gout, w_gloop)


# speedup vs baseline: 2.2602x; 1.2482x over previous
import functools

import jax
import jax.numpy as jnp
from jax import lax
from jax.experimental import pallas as pl
from jax.experimental.pallas import tpu as pltpu
from jax.experimental.pallas import tpu_sc as plsc

N, E, D = 10000, 320000, 128
NC, NS = 2, 16
NW = NC * NS
CH = 128
M = 2 * E
NCHUNK = -(-M // (NW * CH))
M_PAD = NW * CH * NCHUNK
NCH_TOT = M_PAD // CH
R_ACC = 10240
ZROWS = R_ACC // NS

BLK = 1000


def _dense1_body(x_ref, win_ref, wout_ref, wloop_ref, bin_ref, bout_ref,
                 wg_ref, bg_ref, gated_ref, loop_ref):
    xb = x_ref[...]
    wg = wg_ref[...]
    bg = bg_ref[...]

    def gate(k):
        s = jnp.sum(xb * wg[k:k + 1, :], axis=1, keepdims=True) + bg[:, k:k + 1]
        return 1.0 / (1.0 + jnp.exp(-s))

    it = jnp.dot(xb, win_ref[...], preferred_element_type=jnp.float32) + bin_ref[...]
    gated_ref[0] = it * gate(0)
    ot = jnp.dot(xb, wout_ref[...], preferred_element_type=jnp.float32) + bout_ref[...]
    gated_ref[1] = ot * gate(1)
    lt = jnp.dot(xb, wloop_ref[...], preferred_element_type=jnp.float32)
    loop_ref[...] = lt * gate(2)


def _combine_body(p_ref, loop_ref, out_ref):
    out_ref[...] = jnp.maximum(p_ref[0] + p_ref[1] + loop_ref[...], 0.0)


def _sc_scatter_body(table_hbm, gidx_hbm, sidx_hbm, out_hbm,
                     gi0, si0, gi1, si1, rows0, acc, gsem0, isem0, isem1):
    cid = lax.axis_index("c")
    sid = lax.axis_index("s")
    wid = cid * NS + sid
    gi = (gi0, gi1)
    si = (si0, si1)
    isem = (isem0, isem1)

    zero16 = jnp.zeros((16,), jnp.float32)

    def zloop(i, _):
        r = i // 8
        c = (i % 8) * 16
        rows0[r, pl.ds(c, 16)] = zero16
        return 0

    lax.fori_loop(0, CH * 8, zloop, 0)
    for k in range(ZROWS // CH):
        pltpu.sync_copy(rows0, acc.at[pl.ds(sid * ZROWS + k * CH, CH)])
    plsc.subcore_barrier()

    base = wid * NCHUNK

    def idx_issue(c, p):
        pltpu.async_copy(gidx_hbm.at[base + c], gi[p], isem[p])
        pltpu.async_copy(sidx_hbm.at[base + c], si[p], isem[p])

    def idx_wait(p):
        pltpu.make_async_copy(gidx_hbm.at[0], gi[p], isem[p]).wait()
        pltpu.make_async_copy(sidx_hbm.at[0], si[p], isem[p]).wait()

    def step(c, p, prefetch=True):
        idx_wait(p)
        if prefetch:
            idx_issue(c + 1, 1 - p)
        pltpu.async_copy(table_hbm.at[gi[p]], rows0, gsem0).wait()
        pltpu.sync_copy(rows0, acc.at[si[p]], add=True)

    idx_issue(0, 0)

    def pair(k, _):
        step(2 * k, 0)
        step(2 * k + 1, 1)
        return 0

    lax.fori_loop(0, NCHUNK // 2, pair, 0)
    step(NCHUNK - 1, 0, prefetch=False)
    plsc.subcore_barrier()

    pltpu.sync_copy(acc.at[pl.ds(sid * ZROWS, ZROWS)],
                    out_hbm.at[cid, pl.ds(sid * ZROWS, ZROWS)])


@jax.jit
def _run(x, edge_index, w_in, b_in, w_out, b_out, w_loop, w_gin, b_gin,
         w_gout, b_gout, w_gloop):
    src = edge_index[0].astype(jnp.int32)
    dst = edge_index[1].astype(jnp.int32)
    pad = M_PAD - M
    gidx = jnp.concatenate([src, dst + N, jnp.zeros((pad,), jnp.int32)])
    sidx = jnp.concatenate(
        [dst, src, N + (jnp.arange(pad, dtype=jnp.int32) % (R_ACC - N))])
    gidx = gidx.reshape(NCH_TOT, CH)
    sidx = sidx.reshape(NCH_TOT, CH)

    wg = jnp.concatenate([w_gin.T, w_gout.T, w_gloop.T], axis=0)
    bg = jnp.stack([b_gin[0], b_gout[0], jnp.float32(0.0)]).reshape(1, 3)

    gated, loop_act = pl.pallas_call(
        _dense1_body,
        grid=(N // BLK,),
        in_specs=[
            pl.BlockSpec((BLK, D), lambda i: (i, 0)),
            pl.BlockSpec((D, D), lambda i: (0, 0)),
            pl.BlockSpec((D, D), lambda i: (0, 0)),
            pl.BlockSpec((D, D), lambda i: (0, 0)),
            pl.BlockSpec((1, D), lambda i: (0, 0)),
            pl.BlockSpec((1, D), lambda i: (0, 0)),
            pl.BlockSpec((3, D), lambda i: (0, 0)),
            pl.BlockSpec((1, 3), lambda i: (0, 0)),
        ],
        out_specs=[
            pl.BlockSpec((2, BLK, D), lambda i: (0, i, 0)),
            pl.BlockSpec((BLK, D), lambda i: (i, 0)),
        ],
        out_shape=[
            jax.ShapeDtypeStruct((2, N, D), jnp.float32),
            jax.ShapeDtypeStruct((N, D), jnp.float32),
        ],
    )(x, w_in, w_out, w_loop, b_in.reshape(1, D), b_out.reshape(1, D), wg, bg)

    table = gated.reshape(2 * N, D)

    mesh = plsc.VectorSubcoreMesh(core_axis_name="c", subcore_axis_name="s")
    partials = pl.kernel(
        _sc_scatter_body,
        out_type=jax.ShapeDtypeStruct((NC, R_ACC, D), jnp.float32),
        mesh=mesh,
        scratch_types=(
            [pltpu.VMEM((CH,), jnp.int32)] * 4
            + [pltpu.VMEM((CH, D), jnp.float32)]
            + [pltpu.VMEM_SHARED((R_ACC, D), jnp.float32)]
            + [pltpu.SemaphoreType.DMA] * 3
        ),
    )(table, gidx, sidx)

    out = pl.pallas_call(
        _combine_body,
        grid=(N // BLK,),
        in_specs=[
            pl.BlockSpec((2, BLK, D), lambda i: (0, i, 0)),
            pl.BlockSpec((BLK, D), lambda i: (i, 0)),
        ],
        out_specs=pl.BlockSpec((BLK, D), lambda i: (i, 0)),
        out_shape=jax.ShapeDtypeStruct((N, D), jnp.float32),
    )(partials, loop_act)
    return out


def kernel(x, edge_index, w_in, b_in, w_out, b_out, w_loop, w_gin, b_gin,
           w_gout, b_gout, w_gloop):
    return _run(x, edge_index, w_in, b_in, w_out, b_out, w_loop, w_gin, b_gin,
                w_gout, b_gout, w_gloop)


# speedup vs baseline: 2.3557x; 1.0423x over previous
import functools

import jax
import jax.numpy as jnp
from jax import lax
from jax.experimental import pallas as pl
from jax.experimental.pallas import tpu as pltpu
from jax.experimental.pallas import tpu_sc as plsc

N, E, D = 10000, 320000, 128
NC, NS = 2, 16
NW = NC * NS
CH = 128
M = 2 * E
NCHUNK = -(-M // (NW * CH))
M_PAD = NW * CH * NCHUNK
NCH_TOT = M_PAD // CH
R_ACC = 10240
ZROWS = R_ACC // NS

BLK = 1000


def _dense1_body(x_ref, win_ref, wout_ref, bin_ref, bout_ref,
                 wg_ref, bg_ref, gated_ref):
    xb = x_ref[...]
    wg = wg_ref[...]
    bg = bg_ref[...]

    def gate(k):
        s = jnp.sum(xb * wg[k:k + 1, :], axis=1, keepdims=True) + bg[:, k:k + 1]
        return 1.0 / (1.0 + jnp.exp(-s))

    it = jnp.dot(xb, win_ref[...], preferred_element_type=jnp.float32) + bin_ref[...]
    gated_ref[0] = it * gate(0)
    ot = jnp.dot(xb, wout_ref[...], preferred_element_type=jnp.float32) + bout_ref[...]
    gated_ref[1] = ot * gate(1)


def _combine_body(p_ref, x_ref, wloop_ref, wgl_ref, out_ref):
    xb = x_ref[...]
    lt = jnp.dot(xb, wloop_ref[...], preferred_element_type=jnp.float32)
    s = jnp.sum(xb * wgl_ref[...], axis=1, keepdims=True)
    gl = 1.0 / (1.0 + jnp.exp(-s))
    out_ref[...] = jnp.maximum(p_ref[0] + p_ref[1] + lt * gl, 0.0)


def _sc_scatter_body(table_hbm, gidx_hbm, sidx_hbm, out_hbm,
                     gi0, si0, gi1, si1, rows0, acc, gsem0, isem0, isem1):
    cid = lax.axis_index("c")
    sid = lax.axis_index("s")
    wid = cid * NS + sid
    gi = (gi0, gi1)
    si = (si0, si1)
    isem = (isem0, isem1)

    zero16 = jnp.zeros((16,), jnp.float32)

    def zloop(i, _):
        r = i // 8
        c = (i % 8) * 16
        rows0[r, pl.ds(c, 16)] = zero16
        return 0

    lax.fori_loop(0, CH * 8, zloop, 0)
    for k in range(ZROWS // CH):
        pltpu.sync_copy(rows0, acc.at[pl.ds(sid * ZROWS + k * CH, CH)])
    plsc.subcore_barrier()

    base = wid * NCHUNK

    def idx_issue(c, p):
        pltpu.async_copy(gidx_hbm.at[base + c], gi[p], isem[p])
        pltpu.async_copy(sidx_hbm.at[base + c], si[p], isem[p])

    def idx_wait(p):
        pltpu.make_async_copy(gidx_hbm.at[0], gi[p], isem[p]).wait()
        pltpu.make_async_copy(sidx_hbm.at[0], si[p], isem[p]).wait()

    def step(c, p, prefetch=True):
        idx_wait(p)
        if prefetch:
            idx_issue(c + 1, 1 - p)
        pltpu.async_copy(table_hbm.at[gi[p]], rows0, gsem0).wait()
        pltpu.sync_copy(rows0, acc.at[si[p]], add=True)

    idx_issue(0, 0)

    def pair(k, _):
        step(2 * k, 0)
        step(2 * k + 1, 1)
        return 0

    lax.fori_loop(0, NCHUNK // 2, pair, 0)
    step(NCHUNK - 1, 0, prefetch=False)
    plsc.subcore_barrier()

    pltpu.sync_copy(acc.at[pl.ds(sid * ZROWS, ZROWS)],
                    out_hbm.at[cid, pl.ds(sid * ZROWS, ZROWS)])


@jax.jit
def _run(x, edge_index, w_in, b_in, w_out, b_out, w_loop, w_gin, b_gin,
         w_gout, b_gout, w_gloop):
    src = edge_index[0].astype(jnp.int32)
    dst = edge_index[1].astype(jnp.int32)
    pad = M_PAD - M
    gidx = jnp.concatenate([src, dst + N, jnp.zeros((pad,), jnp.int32)])
    sidx = jnp.concatenate(
        [dst, src, N + (jnp.arange(pad, dtype=jnp.int32) % (R_ACC - N))])
    gidx = gidx.reshape(NCH_TOT, CH)
    sidx = sidx.reshape(NCH_TOT, CH)

    wg = jnp.concatenate([w_gin.T, w_gout.T], axis=0)
    bg = jnp.stack([b_gin[0], b_gout[0]]).reshape(1, 2)

    gated = pl.pallas_call(
        _dense1_body,
        grid=(N // BLK,),
        in_specs=[
            pl.BlockSpec((BLK, D), lambda i: (i, 0)),
            pl.BlockSpec((D, D), lambda i: (0, 0)),
            pl.BlockSpec((D, D), lambda i: (0, 0)),
            pl.BlockSpec((1, D), lambda i: (0, 0)),
            pl.BlockSpec((1, D), lambda i: (0, 0)),
            pl.BlockSpec((2, D), lambda i: (0, 0)),
            pl.BlockSpec((1, 2), lambda i: (0, 0)),
        ],
        out_specs=pl.BlockSpec((2, BLK, D), lambda i: (0, i, 0)),
        out_shape=jax.ShapeDtypeStruct((2, N, D), jnp.float32),
    )(x, w_in, w_out, b_in.reshape(1, D), b_out.reshape(1, D), wg, bg)

    table = gated.reshape(2 * N, D)

    mesh = plsc.VectorSubcoreMesh(core_axis_name="c", subcore_axis_name="s")
    partials = pl.kernel(
        _sc_scatter_body,
        out_type=jax.ShapeDtypeStruct((NC, R_ACC, D), jnp.float32),
        mesh=mesh,
        scratch_types=(
            [pltpu.VMEM((CH,), jnp.int32)] * 4
            + [pltpu.VMEM((CH, D), jnp.float32)]
            + [pltpu.VMEM_SHARED((R_ACC, D), jnp.float32)]
            + [pltpu.SemaphoreType.DMA] * 3
        ),
    )(table, gidx, sidx)

    out = pl.pallas_call(
        _combine_body,
        grid=(N // BLK,),
        in_specs=[
            pl.BlockSpec((2, BLK, D), lambda i: (0, i, 0)),
            pl.BlockSpec((BLK, D), lambda i: (i, 0)),
            pl.BlockSpec((D, D), lambda i: (0, 0)),
            pl.BlockSpec((1, D), lambda i: (0, 0)),
        ],
        out_specs=pl.BlockSpec((BLK, D), lambda i: (i, 0)),
        out_shape=jax.ShapeDtypeStruct((N, D), jnp.float32),
    )(partials, x, w_loop, w_gloop.T)
    return out


def kernel(x, edge_index, w_in, b_in, w_out, b_out, w_loop, w_gin, b_gin,
           w_gout, b_gout, w_gloop):
    return _run(x, edge_index, w_in, b_in, w_out, b_out, w_loop, w_gin, b_gin,
                w_gout, b_gout, w_gloop)
